# Initial kernel scaffold; baseline (speedup 1.0000x reference)
import jax
import jax.numpy as jnp
from jax.experimental import pallas as pl


def kernel(nfeats, efeats, edge_index, W_w, W_b, A_w, A_b):
    raise NotImplementedError("write your pallas kernel here")


# speedup vs baseline: 24.5943x; 24.5943x over previous
import functools

import jax
import jax.numpy as jnp
from jax import lax
from jax.experimental import pallas as pl
from jax.experimental.pallas import tpu as pltpu
from jax.experimental.pallas import tpu_sc as plsc

_N = 10000
_E = 320000
_DIN = 128
_DE = 16
_DOUT = 128

_NC, _NS, _L = 2, 16, 16
_EPT = _E // (_NC * _NS)
_EPC = _E // _NC
_SUB = 2000
_NSUB = _EPT // _SUB
_ROWS = _E // _SUB
_NP = 10240
_RPT = _NP // _NS


def _edge_scores(ssrc_v, sdst_v, src_ref, dst_ref, ee_ref, n):
  def body(i, carry):
    sl = pl.ds(i * _L, _L)
    vs = plsc.load_gather(ssrc_v, [src_ref[sl]])
    vd = plsc.load_gather(sdst_v, [dst_ref[sl]])
    ee_ref[sl] = jnp.exp(jnp.maximum(vs + vd, 0.0))
    return carry
  lax.fori_loop(0, n // _L, body, 0)


def _gat_sc(ssrc_hbm, sdst_hbm, src_hbm, dst_hbm, ef_hbm, z1d_hbm, z2d_hbm,
            zpart_hbm,
            ssrc_v, sdst_v, srcA_v, dstA_v, eexpA_v, srcB_v, dstB_v, eexpB_v,
            denom_v, rows_v, denom_sh, z_sh):
  c = lax.axis_index("c")
  s = lax.axis_index("s")
  r0 = s * _RPT

  pltpu.sync_copy(z1d_hbm.at[pl.ds(r0, _RPT)], denom_sh.at[pl.ds(r0, _RPT)])
  pltpu.sync_copy(z2d_hbm.at[pl.ds(r0, _RPT)], z_sh.at[pl.ds(r0, _RPT)])

  pltpu.sync_copy(ssrc_hbm, ssrc_v)
  pltpu.sync_copy(sdst_hbm, sdst_v)
  plsc.subcore_barrier()

  ofsA = c * _EPC + s * _EPT
  ofsB = (1 - c) * _EPC + s * _EPT

  pltpu.sync_copy(src_hbm.at[pl.ds(ofsA, _EPT)], srcA_v)
  pltpu.sync_copy(dst_hbm.at[pl.ds(ofsA, _EPT)], dstA_v)
  _edge_scores(ssrc_v, sdst_v, srcA_v, dstA_v, eexpA_v, _EPT)
  pltpu.sync_copy(eexpA_v, denom_sh.at[dstA_v], add=True)

  for j in range(_NSUB):
    pltpu.sync_copy(src_hbm.at[pl.ds(ofsB + j * _SUB, _SUB)], srcB_v)
    pltpu.sync_copy(dst_hbm.at[pl.ds(ofsB + j * _SUB, _SUB)], dstB_v)
    _edge_scores(ssrc_v, sdst_v, srcB_v, dstB_v, eexpB_v, _SUB)
    pltpu.sync_copy(eexpB_v, denom_sh.at[dstB_v], add=True)

  plsc.subcore_barrier()

  pltpu.sync_copy(denom_sh, denom_v)
  def abody(i, carry):
    sl = pl.ds(i * _L, _L)
    dn = plsc.load_gather(denom_v, [dstA_v[sl]])
    eexpA_v[sl] = eexpA_v[sl] / dn
    return carry
  lax.fori_loop(0, _EPT // _L, abody, 0)

  for j in range(_NSUB):
    pltpu.sync_copy(ef_hbm.at[pl.ds(ofsA + j * _SUB, _SUB)], rows_v)

    def sbody(g, carry, j=j):
      a = eexpA_v[pl.ds(j * _SUB + g * _L, _L)]
      for k in range(_L):
        e = g * _L + k
        rows_v[e, :] = rows_v[e, :] * a[k]
      return carry
    lax.fori_loop(0, _SUB // _L, sbody, 0)

    pltpu.sync_copy(rows_v, z_sh.at[dstA_v.at[pl.ds(j * _SUB, _SUB)]], add=True)

  plsc.subcore_barrier()
  pltpu.sync_copy(z_sh.at[pl.ds(r0, _RPT)], zpart_hbm.at[c, pl.ds(r0, _RPT)])


_gat_sc_call = functools.partial(
    pl.kernel,
    out_type=jax.ShapeDtypeStruct((_NC, _NP, _DE), jnp.float32),
    mesh=plsc.VectorSubcoreMesh(core_axis_name="c", subcore_axis_name="s",
                                num_cores=_NC, num_subcores=_NS),
    compiler_params=pltpu.CompilerParams(needs_layout_passes=False,
                                         use_tc_tiling_on_sc=False),
    scratch_types=[
        pltpu.VMEM((_N,), jnp.float32),
        pltpu.VMEM((_N,), jnp.float32),
        pltpu.VMEM((_EPT,), jnp.int32),
        pltpu.VMEM((_EPT,), jnp.int32),
        pltpu.VMEM((_EPT,), jnp.float32),
        pltpu.VMEM((_SUB,), jnp.int32),
        pltpu.VMEM((_SUB,), jnp.int32),
        pltpu.VMEM((_SUB,), jnp.float32),
        pltpu.VMEM((_NP,), jnp.float32),
        pltpu.VMEM((_SUB, _DE), jnp.float32),
        pltpu.VMEM_SHARED((_NP,), jnp.float32),
        pltpu.VMEM_SHARED((_NP, _DE), jnp.float32),
    ],
)(_gat_sc)


def _s_tc(x_ref, a_ref, b_ref, o_ref):
  o_ref[...] = jnp.dot(x_ref[...], a_ref[...],
                       preferred_element_type=jnp.float32) + b_ref[...]


def _out_tc(x_ref, za_ref, zb_ref, w1_ref, w2_ref, b_ref, o_ref):
  z = za_ref[...] + zb_ref[...]
  acc = jnp.dot(x_ref[...], w1_ref[...], preferred_element_type=jnp.float32)
  acc += jnp.dot(z, w2_ref[...], preferred_element_type=jnp.float32)
  o_ref[...] = jnp.maximum(acc + b_ref[...], 0.0)


def kernel(nfeats, efeats, edge_index, W_w, W_b, A_w, A_b):
  X = nfeats.reshape(_N, _DIN).astype(jnp.float32)
  ef = efeats.reshape(_E, _DE).astype(jnp.float32)
  src = edge_index[0].astype(jnp.int32)
  dst = edge_index[1].astype(jnp.int32)

  A2 = jnp.zeros((_DIN, 128), jnp.float32)
  A2 = A2.at[:, 0].set(A_w[0, :_DIN]).at[:, 1].set(A_w[0, _DIN:])
  b2 = jnp.zeros((1, 128), jnp.float32).at[0, 1].set(A_b[0])
  sp = pl.pallas_call(
      _s_tc,
      grid=(5,),
      in_specs=[
          pl.BlockSpec((_N // 5, _DIN), lambda i: (i, 0)),
          pl.BlockSpec((_DIN, 128), lambda i: (0, 0)),
          pl.BlockSpec((1, 128), lambda i: (0, 0)),
      ],
      out_specs=pl.BlockSpec((_N // 5, 128), lambda i: (i, 0)),
      out_shape=jax.ShapeDtypeStruct((_N, 128), jnp.float32),
  )(X, A2, b2)
  s_src = sp[:, 0]
  s_dst = sp[:, 1]

  zpart = _gat_sc_call(s_src, s_dst, src, dst, ef,
                       jnp.zeros((_NP,), jnp.float32),
                       jnp.zeros((_NP, _DE), jnp.float32))

  out = pl.pallas_call(
      _out_tc,
      grid=(10,),
      in_specs=[
          pl.BlockSpec((_N // 10, _DIN), lambda i: (i, 0)),
          pl.BlockSpec((_N // 10, _DE), lambda i: (i, 0)),
          pl.BlockSpec((_N // 10, _DE), lambda i: (i, 0)),
          pl.BlockSpec((_DIN, _DOUT), lambda i: (0, 0)),
          pl.BlockSpec((_DE, _DOUT), lambda i: (0, 0)),
          pl.BlockSpec((1, _DOUT), lambda i: (0, 0)),
      ],
      out_specs=pl.BlockSpec((_N // 10, _DOUT), lambda i: (i, 0)),
      out_shape=jax.ShapeDtypeStruct((_N, _DOUT), jnp.float32),
  )(X, zpart[0, :_N], zpart[1, :_N], W_w[:, :_DIN].T, W_w[:, _DIN:].T,
    W_b.reshape(1, _DOUT))
  return out.reshape(_N, 1, _DOUT)


# speedup vs baseline: 29.7746x; 1.2106x over previous
import functools

import jax
import jax.numpy as jnp
from jax import lax
from jax.experimental import pallas as pl
from jax.experimental.pallas import tpu as pltpu
from jax.experimental.pallas import tpu_sc as plsc

_N = 10000
_E = 320000
_DIN = 128
_DE = 16
_DOUT = 128

_NC, _NS, _L = 2, 16, 16
_EPT = _E // (_NC * _NS)
_EPC = _E // _NC
_SUB = 2000
_NSUB = _EPT // _SUB
_NP = 10240
_RPT = _NP // _NS

_SC_PARAMS = pltpu.CompilerParams(needs_layout_passes=False,
                                  use_tc_tiling_on_sc=False)
_MESH = plsc.VectorSubcoreMesh(core_axis_name="c", subcore_axis_name="s",
                               num_cores=_NC, num_subcores=_NS)


def _sc_denom(sp8_hbm, ei_hbm, z1d_hbm, eexp_hbm, dpart_hbm,
              ssrc_v, sdst_v, src_v, dst_v, eexp_v, denom_sh):
  c = lax.axis_index("c")
  s = lax.axis_index("s")
  r0 = s * _RPT
  ofs = c * _EPC + s * _EPT

  pltpu.sync_copy(z1d_hbm.at[pl.ds(r0, _RPT)], denom_sh.at[pl.ds(r0, _RPT)])
  pltpu.sync_copy(sp8_hbm.at[0], ssrc_v)
  pltpu.sync_copy(sp8_hbm.at[1], sdst_v)
  pltpu.sync_copy(ei_hbm.at[0, pl.ds(ofs, _EPT)], src_v)
  pltpu.sync_copy(ei_hbm.at[1, pl.ds(ofs, _EPT)], dst_v)
  plsc.subcore_barrier()

  def body(i, carry):
    sl = pl.ds(i * _L, _L)
    vs = plsc.load_gather(ssrc_v, [src_v[sl]])
    vd = plsc.load_gather(sdst_v, [dst_v[sl]])
    eexp_v[sl] = jnp.exp(jnp.maximum(vs + vd, 0.0))
    return carry
  lax.fori_loop(0, _EPT // _L, body, 0)

  pltpu.sync_copy(eexp_v, eexp_hbm.at[pl.ds(ofs, _EPT)])
  pltpu.sync_copy(eexp_v, denom_sh.at[dst_v], add=True)
  plsc.subcore_barrier()
  pltpu.sync_copy(denom_sh.at[pl.ds(r0, _RPT)], dpart_hbm.at[c, pl.ds(r0, _RPT)])


_sc_denom_call = functools.partial(
    pl.kernel,
    out_type=(jax.ShapeDtypeStruct((_E,), jnp.float32),
              jax.ShapeDtypeStruct((_NC, _NP), jnp.float32)),
    mesh=_MESH,
    compiler_params=_SC_PARAMS,
    scratch_types=[
        pltpu.VMEM((_N,), jnp.float32),
        pltpu.VMEM((_N,), jnp.float32),
        pltpu.VMEM((_EPT,), jnp.int32),
        pltpu.VMEM((_EPT,), jnp.int32),
        pltpu.VMEM((_EPT,), jnp.float32),
        pltpu.VMEM_SHARED((_NP,), jnp.float32),
    ],
)(_sc_denom)


def _sc_zsum(ei_hbm, eexp_hbm, dpart_hbm, ef_hbm, z2d_hbm,
             zpart_hbm,
             dst_v, alpha_v, d0_v, d1_v, rows_v, z_sh):
  c = lax.axis_index("c")
  s = lax.axis_index("s")
  r0 = s * _RPT
  ofs = c * _EPC + s * _EPT

  pltpu.sync_copy(z2d_hbm.at[pl.ds(r0, _RPT)], z_sh.at[pl.ds(r0, _RPT)])
  pltpu.sync_copy(dpart_hbm.at[0], d0_v)
  pltpu.sync_copy(dpart_hbm.at[1], d1_v)
  pltpu.sync_copy(ei_hbm.at[1, pl.ds(ofs, _EPT)], dst_v)
  pltpu.sync_copy(eexp_hbm.at[pl.ds(ofs, _EPT)], alpha_v)
  plsc.subcore_barrier()

  def abody(i, carry):
    sl = pl.ds(i * _L, _L)
    di = dst_v[sl]
    dn = plsc.load_gather(d0_v, [di]) + plsc.load_gather(d1_v, [di])
    alpha_v[sl] = alpha_v[sl] / dn
    return carry
  lax.fori_loop(0, _EPT // _L, abody, 0)

  for j in range(_NSUB):
    pltpu.sync_copy(ef_hbm.at[pl.ds(ofs + j * _SUB, _SUB)], rows_v)

    def sbody(g, carry, j=j):
      a = alpha_v[pl.ds(j * _SUB + g * _L, _L)]
      for k in range(_L):
        e = g * _L + k
        rows_v[e, :] = rows_v[e, :] * a[k]
      return carry
    lax.fori_loop(0, _SUB // _L, sbody, 0)

    pltpu.sync_copy(rows_v, z_sh.at[dst_v.at[pl.ds(j * _SUB, _SUB)]], add=True)

  plsc.subcore_barrier()
  pltpu.sync_copy(z_sh.at[pl.ds(r0, _RPT)], zpart_hbm.at[c, pl.ds(r0, _RPT)])


_sc_zsum_call = functools.partial(
    pl.kernel,
    out_type=jax.ShapeDtypeStruct((_NC, _NP, _DE), jnp.float32),
    mesh=_MESH,
    compiler_params=_SC_PARAMS,
    scratch_types=[
        pltpu.VMEM((_EPT,), jnp.int32),
        pltpu.VMEM((_EPT,), jnp.float32),
        pltpu.VMEM((_NP,), jnp.float32),
        pltpu.VMEM((_NP,), jnp.float32),
        pltpu.VMEM((_SUB, _DE), jnp.float32),
        pltpu.VMEM_SHARED((_NP, _DE), jnp.float32),
    ],
)(_sc_zsum)


def _s_tc(x_ref, a_ref, b_ref, o_ref):
  x = x_ref[...].reshape(x_ref.shape[0], _DIN)
  o_ref[...] = lax.dot_general(a_ref[...], x, (((1,), (1,)), ((), ())),
                               preferred_element_type=jnp.float32) + b_ref[...]


def _out_tc(x_ref, za_ref, zb_ref, w1_ref, w2_ref, b_ref, o_ref):
  nb = x_ref.shape[0]
  x = x_ref[...].reshape(nb, _DIN)
  z = za_ref[...] + zb_ref[...]
  acc = jnp.dot(x, w1_ref[...], preferred_element_type=jnp.float32)
  acc += jnp.dot(z, w2_ref[...], preferred_element_type=jnp.float32)
  o_ref[...] = jnp.maximum(acc + b_ref[...], 0.0).reshape(nb, 1, _DOUT)


def kernel(nfeats, efeats, edge_index, W_w, W_b, A_w, A_b):
  ef = efeats.reshape(_E, _DE).astype(jnp.float32)
  ei = edge_index.astype(jnp.int32)

  A8 = jnp.zeros((8, _DIN), jnp.float32)
  A8 = A8.at[0].set(A_w[0, :_DIN]).at[1].set(A_w[0, _DIN:])
  sp8 = pl.pallas_call(
      _s_tc,
      out_shape=jax.ShapeDtypeStruct((8, _N), jnp.float32),
  )(nfeats, A8, jnp.zeros((8, 1), jnp.float32).at[1, 0].set(A_b[0]))

  eexp, dpart = _sc_denom_call(sp8, ei, jnp.zeros((_NP,), jnp.float32))
  zpart = _sc_zsum_call(ei, eexp, dpart, ef,
                        jnp.zeros((_NP, _DE), jnp.float32))

  out = pl.pallas_call(
      _out_tc,
      grid=(10,),
      in_specs=[
          pl.BlockSpec((_N // 10, 1, _DIN), lambda i: (i, 0, 0)),
          pl.BlockSpec((_N // 10, _DE), lambda i: (i, 0)),
          pl.BlockSpec((_N // 10, _DE), lambda i: (i, 0)),
          pl.BlockSpec((_DIN, _DOUT), lambda i: (0, 0)),
          pl.BlockSpec((_DE, _DOUT), lambda i: (0, 0)),
          pl.BlockSpec((1, _DOUT), lambda i: (0, 0)),
      ],
      out_specs=pl.BlockSpec((_N // 10, 1, _DOUT), lambda i: (i, 0, 0)),
      out_shape=jax.ShapeDtypeStruct((_N, 1, _DOUT), jnp.float32),
  )(nfeats, zpart[0, :_N], zpart[1, :_N], W_w[:, :_DIN].T, W_w[:, _DIN:].T,
    W_b.reshape(1, _DOUT))
  return out


# speedup vs baseline: 32.7924x; 1.1014x over previous
import functools

import jax
import jax.numpy as jnp
from jax import lax
from jax.experimental import pallas as pl
from jax.experimental.pallas import tpu as pltpu
from jax.experimental.pallas import tpu_sc as plsc

_N = 10000
_E = 320000
_DIN = 128
_DE = 16
_DOUT = 128

_NC, _NS, _L = 2, 16, 16
_EPT = _E // (_NC * _NS)
_EPC = _E // _NC
_SUB = 2000
_NSUB = _EPT // _SUB
_NP = 10240
_RPT = _NP // _NS

_SC_PARAMS = pltpu.CompilerParams(needs_layout_passes=False,
                                  use_tc_tiling_on_sc=False)
_MESH = plsc.VectorSubcoreMesh(core_axis_name="c", subcore_axis_name="s",
                               num_cores=_NC, num_subcores=_NS)


def _sc_denom(sp8_hbm, ei_hbm, z1d_hbm, eexp_hbm, dpart_hbm,
              ssrc_v, sdst_v, src_v, dst_v, eexp_v, denom_sh):
  c = lax.axis_index("c")
  s = lax.axis_index("s")
  r0 = s * _RPT
  ofs = c * _EPC + s * _EPT

  pltpu.sync_copy(z1d_hbm.at[pl.ds(r0, _RPT)], denom_sh.at[pl.ds(r0, _RPT)])
  pltpu.sync_copy(sp8_hbm.at[0], ssrc_v)
  pltpu.sync_copy(sp8_hbm.at[1], sdst_v)
  pltpu.sync_copy(ei_hbm.at[0, pl.ds(ofs, _EPT)], src_v)
  pltpu.sync_copy(ei_hbm.at[1, pl.ds(ofs, _EPT)], dst_v)
  plsc.subcore_barrier()

  def body(i, carry):
    sl = pl.ds(i * _L, _L)
    vs = plsc.load_gather(ssrc_v, [src_v[sl]])
    vd = plsc.load_gather(sdst_v, [dst_v[sl]])
    eexp_v[sl] = jnp.exp(jnp.maximum(vs + vd, 0.0))
    return carry
  lax.fori_loop(0, _EPT // _L, body, 0)

  pltpu.sync_copy(eexp_v, eexp_hbm.at[pl.ds(ofs, _EPT)])
  pltpu.sync_copy(eexp_v, denom_sh.at[dst_v], add=True)
  plsc.subcore_barrier()
  pltpu.sync_copy(denom_sh.at[pl.ds(r0, _RPT)], dpart_hbm.at[c, pl.ds(r0, _RPT)])


_sc_denom_call = functools.partial(
    pl.kernel,
    out_type=(jax.ShapeDtypeStruct((_E,), jnp.float32),
              jax.ShapeDtypeStruct((_NC, _NP), jnp.float32)),
    mesh=_MESH,
    compiler_params=_SC_PARAMS,
    scratch_types=[
        pltpu.VMEM((_N,), jnp.float32),
        pltpu.VMEM((_N,), jnp.float32),
        pltpu.VMEM((_EPT,), jnp.int32),
        pltpu.VMEM((_EPT,), jnp.int32),
        pltpu.VMEM((_EPT,), jnp.float32),
        pltpu.VMEM_SHARED((_NP,), jnp.float32),
    ],
)(_sc_denom)


def _sc_zsum(ei_hbm, eexp_hbm, ef_hbm, z2d_hbm,
             zpart_hbm,
             dst_v, eexp_v, rows0_v, rows1_v, z_sh, lsem, ssem):
  c = lax.axis_index("c")
  s = lax.axis_index("s")
  r0 = s * _RPT
  ofs = c * _EPC + s * _EPT
  bufs = (rows0_v, rows1_v)

  cz = pltpu.async_copy(z2d_hbm.at[pl.ds(r0, _RPT)], z_sh.at[pl.ds(r0, _RPT)],
                        lsem)
  cd = pltpu.async_copy(ei_hbm.at[1, pl.ds(ofs, _EPT)], dst_v, lsem)
  ce = pltpu.async_copy(eexp_hbm.at[pl.ds(ofs, _EPT)], eexp_v, lsem)
  loads = [pltpu.async_copy(ef_hbm.at[pl.ds(ofs, _SUB)], rows0_v, lsem)]
  cz.wait()
  cd.wait()
  ce.wait()
  plsc.subcore_barrier()

  scatters = []
  for j in range(_NSUB):
    buf = bufs[j % 2]
    loads[j].wait()
    if j >= 1:
      scatters[j - 1].wait()
    if j + 1 < _NSUB:
      loads.append(pltpu.async_copy(
          ef_hbm.at[pl.ds(ofs + (j + 1) * _SUB, _SUB)], bufs[(j + 1) % 2],
          lsem))

    def sbody(g, carry, j=j, buf=buf):
      a = eexp_v[pl.ds(j * _SUB + g * _L, _L)]
      for k in range(_L):
        e = g * _L + k
        buf[e, :] = buf[e, :] * a[k]
      return carry
    lax.fori_loop(0, _SUB // _L, sbody, 0)

    scatters.append(pltpu.async_copy(
        buf, z_sh.at[dst_v.at[pl.ds(j * _SUB, _SUB)]], ssem, add=True))

  scatters[-1].wait()
  plsc.subcore_barrier()
  pltpu.sync_copy(z_sh.at[pl.ds(r0, _RPT)], zpart_hbm.at[c, pl.ds(r0, _RPT)])


_sc_zsum_call = functools.partial(
    pl.kernel,
    out_type=jax.ShapeDtypeStruct((_NC, _NP, _DE), jnp.float32),
    mesh=_MESH,
    compiler_params=_SC_PARAMS,
    scratch_types=[
        pltpu.VMEM((_EPT,), jnp.int32),
        pltpu.VMEM((_EPT,), jnp.float32),
        pltpu.VMEM((_SUB, _DE), jnp.float32),
        pltpu.VMEM((_SUB, _DE), jnp.float32),
        pltpu.VMEM_SHARED((_NP, _DE), jnp.float32),
        pltpu.SemaphoreType.DMA,
        pltpu.SemaphoreType.DMA,
    ],
)(_sc_zsum)


def _s_tc(x_ref, a_ref, b_ref, o_ref):
  x = x_ref[...].reshape(x_ref.shape[0], _DIN)
  o_ref[...] = lax.dot_general(a_ref[...], x, (((1,), (1,)), ((), ())),
                               preferred_element_type=jnp.float32) + b_ref[...]


def _out_tc(x_ref, za_ref, zb_ref, d0_ref, d1_ref, w1_ref, w2_ref, b_ref,
            o_ref):
  nb = x_ref.shape[0]
  x = x_ref[...].reshape(nb, _DIN)
  d = d0_ref[...] + d1_ref[...]
  z = (za_ref[...] + zb_ref[...]) * jnp.where(d > 0.0, 1.0 / d, 0.0)
  acc = jnp.dot(x, w1_ref[...], preferred_element_type=jnp.float32)
  acc += jnp.dot(z, w2_ref[...], preferred_element_type=jnp.float32)
  o_ref[...] = jnp.maximum(acc + b_ref[...], 0.0).reshape(nb, 1, _DOUT)


def kernel(nfeats, efeats, edge_index, W_w, W_b, A_w, A_b):
  ef = efeats.reshape(_E, _DE).astype(jnp.float32)
  ei = edge_index.astype(jnp.int32)

  A8 = jnp.zeros((8, _DIN), jnp.float32)
  A8 = A8.at[0].set(A_w[0, :_DIN]).at[1].set(A_w[0, _DIN:])
  sp8 = pl.pallas_call(
      _s_tc,
      out_shape=jax.ShapeDtypeStruct((8, _N), jnp.float32),
  )(nfeats, A8, jnp.zeros((8, 1), jnp.float32).at[1, 0].set(A_b[0]))

  eexp, dpart = _sc_denom_call(sp8, ei, jnp.zeros((_NP,), jnp.float32))
  zpart = _sc_zsum_call(ei, eexp, ef, jnp.zeros((_NP, _DE), jnp.float32))

  out = pl.pallas_call(
      _out_tc,
      grid=(10,),
      in_specs=[
          pl.BlockSpec((_N // 10, 1, _DIN), lambda i: (i, 0, 0)),
          pl.BlockSpec((_N // 10, _DE), lambda i: (i, 0)),
          pl.BlockSpec((_N // 10, _DE), lambda i: (i, 0)),
          pl.BlockSpec((_N // 10, 1), lambda i: (i, 0)),
          pl.BlockSpec((_N // 10, 1), lambda i: (i, 0)),
          pl.BlockSpec((_DIN, _DOUT), lambda i: (0, 0)),
          pl.BlockSpec((_DE, _DOUT), lambda i: (0, 0)),
          pl.BlockSpec((1, _DOUT), lambda i: (0, 0)),
      ],
      out_specs=pl.BlockSpec((_N // 10, 1, _DOUT), lambda i: (i, 0, 0)),
      out_shape=jax.ShapeDtypeStruct((_N, 1, _DOUT), jnp.float32),
  )(nfeats, zpart[0, :_N], zpart[1, :_N],
    dpart[0, :_N].reshape(_N, 1), dpart[1, :_N].reshape(_N, 1),
    W_w[:, :_DIN].T, W_w[:, _DIN:].T, W_b.reshape(1, _DOUT))
  return out
